# baseline (speedup 1.0000x reference)
import functools

import jax
import jax.numpy as jnp
from jax import lax
from jax.experimental import pallas as pl
from jax.experimental.pallas import tpu as pltpu
from jax.experimental.pallas import tpu_sc as plsc

_B, _T, _D, _E, _TOPK = 4, 4096, 4096, 64, 2
_BT = _B * _T
_TM = 1024
_NW = 32
_CT = _BT // _NW
_L = 16
_UNROLL = 8


def _gate_softmax_block(x_ref, w_ref, probs_ref):
    x = x_ref[...]
    w = w_ref[...]
    logits = lax.dot_general(x, w, (((1,), (1,)), ((), ())),
                             preferred_element_type=jnp.float32)
    m = jnp.max(logits, axis=-1, keepdims=True)
    ex = jnp.exp(logits - m)
    probs_ref[...] = ex / jnp.sum(ex, axis=-1, keepdims=True)


def _gate_softmax(x2, W):
    return pl.pallas_call(
        _gate_softmax_block,
        grid=(_BT // _TM,),
        in_specs=[
            pl.BlockSpec((_TM, _D), lambda i: (i, 0)),
            pl.BlockSpec((_E, _D), lambda i: (0, 0)),
        ],
        out_specs=pl.BlockSpec((_TM, _E), lambda i: (i, 0)),
        out_shape=jax.ShapeDtypeStruct((_BT, _E), jnp.float32),
    )(x2, W)


def _top2_body(probs_hbm, idx_hbm, wts_hbm, p_v, i_v, w_v):
    wid = lax.axis_index("s") * 2 + lax.axis_index("c")
    base = wid * _CT
    pltpu.sync_copy(probs_hbm.at[pl.ds(base * _E, _CT * _E)], p_v)

    lanes = lax.iota(jnp.int32, _L)

    def group_body(g, _):
        tok = g * _L + lanes
        m1 = jnp.full((_L,), -1.0, jnp.float32)
        m2 = jnp.full((_L,), -1.0, jnp.float32)
        i1 = jnp.zeros((_L,), jnp.int32)
        i2 = jnp.zeros((_L,), jnp.int32)
        rowbase = tok * _E

        def expert_body(eb, carry):
            m1, m2, i1, i2 = carry
            for u in range(_UNROLL):
                e = eb * _UNROLL + u
                ev = jnp.full((_L,), e, jnp.int32)
                p = plsc.load_gather(p_v, [rowbase + e])
                gt1 = p > m1
                gt2 = p > m2
                i2 = jnp.where(gt1, i1, jnp.where(gt2, ev, i2))
                m2 = jnp.where(gt1, m1, jnp.where(gt2, p, m2))
                i1 = jnp.where(gt1, ev, i1)
                m1 = jnp.where(gt1, p, m1)
            return m1, m2, i1, i2

        m1, m2, i1, i2 = lax.fori_loop(0, _E // _UNROLL, expert_body,
                                       (m1, m2, i1, i2))
        s = m1 + m2
        out2 = tok * _TOPK
        plsc.store_scatter(i_v, [out2], i1)
        plsc.store_scatter(i_v, [out2 + 1], i2)
        plsc.store_scatter(w_v, [out2], m1 / s)
        plsc.store_scatter(w_v, [out2 + 1], m2 / s)
        return 0

    lax.fori_loop(0, _CT // _L, group_body, 0)
    pltpu.sync_copy(i_v, idx_hbm.at[pl.ds(base * _TOPK, _CT * _TOPK)])
    pltpu.sync_copy(w_v, wts_hbm.at[pl.ds(base * _TOPK, _CT * _TOPK)])


def _top2_sc(probs):
    mesh = plsc.VectorSubcoreMesh(core_axis_name="c", subcore_axis_name="s")
    k = functools.partial(
        pl.kernel,
        mesh=mesh,
        out_type=[
            jax.ShapeDtypeStruct((_BT * _TOPK,), jnp.int32),
            jax.ShapeDtypeStruct((_BT * _TOPK,), jnp.float32),
        ],
        scratch_types=[
            pltpu.VMEM((_CT * _E,), jnp.float32),
            pltpu.VMEM((_CT * _TOPK,), jnp.int32),
            pltpu.VMEM((_CT * _TOPK,), jnp.float32),
        ],
        compiler_params=pltpu.CompilerParams(needs_layout_passes=False),
    )(_top2_body)
    return k(probs.reshape(_BT * _E))


def kernel(x, W):
    x2 = x.reshape(_BT, _D)
    probs = _gate_softmax(x2, W)
    idx, wts = _top2_sc(probs)
    return (probs.reshape(_B, _T, _E),
            idx.reshape(_B, _T, _TOPK),
            wts.reshape(_B, _T, _TOPK))
